# baseline (speedup 1.0000x reference)
import functools

import jax
import jax.numpy as jnp
from jax import lax
from jax.experimental import pallas as pl
from jax.experimental.pallas import tpu as pltpu
from jax.experimental.pallas import tpu_sc as plsc

N = 10000
DEG = 32
D = 128

NW = 32
RPW = 336
NPAD = NW * RPW
C = 4
NCHUNK = RPW // C
NBUF = 3
PER_W = (RPW + NBUF * C) * DEG

_MESH = plsc.VectorSubcoreMesh(core_axis_name="c", subcore_axis_name="s")


@functools.partial(
    pl.kernel,
    mesh=_MESH,
    out_type=jax.ShapeDtypeStruct((NPAD, D), jnp.float32),
    scratch_types=[
        pltpu.VMEM((PER_W,), jnp.int32),
        pltpu.VMEM((NCHUNK, C * DEG), jnp.int32),
        pltpu.VMEM((NBUF, C * DEG, D), jnp.float32),
        pltpu.VMEM_SHARED((16 * RPW, D), jnp.float32),
        pltpu.SemaphoreType.DMA((NBUF,)),
        pltpu.SemaphoreType.DMA((NBUF,)),
    ],
)
def _sc_gather_sum(table, idx_hbm, accidx_hbm, out_hbm,
                   idx_v, accidx_v, rows_v, acc_sh, gsem, ssem):
    sid = lax.axis_index("s")
    wid = sid * 2 + lax.axis_index("c")
    base = wid * RPW
    pltpu.sync_copy(idx_hbm.at[wid], idx_v)
    pltpu.sync_copy(accidx_hbm.at[sid], accidx_v)

    def zero_row(i, carry):
        z = jnp.zeros((16,), jnp.float32)
        for j in range(D // 16):
            rows_v[0, i, pl.ds(j * 16, 16)] = z
        return carry

    lax.fori_loop(0, C * DEG, zero_row, 0)
    for t in range(RPW // (C * DEG)):
        pltpu.sync_copy(
            rows_v.at[0], acc_sh.at[pl.ds(sid * RPW + t * (C * DEG), C * DEG)]
        )
    _REM = RPW % (C * DEG)
    if _REM:
        pltpu.sync_copy(
            rows_v.at[0].at[pl.ds(0, _REM)],
            acc_sh.at[pl.ds(sid * RPW + RPW - _REM, _REM)],
        )

    def start_gather(ci, b):
        pltpu.async_copy(
            table.at[idx_v.at[pl.ds(ci * (C * DEG), C * DEG)]],
            rows_v.at[b],
            gsem.at[b],
        )

    def wait_gather(b):
        pltpu.make_async_copy(
            table.at[idx_v.at[pl.ds(0, C * DEG)]], rows_v.at[b], gsem.at[b]
        ).wait()

    for b in range(NBUF):
        start_gather(b, b)

    def group(gi, carry):
        for b in range(NBUF):
            ci = gi * NBUF + b
            wait_gather(b)
            pltpu.async_copy(
                rows_v.at[b], acc_sh.at[accidx_v.at[ci]], ssem.at[b], add=True
            ).wait()
            start_gather(ci + NBUF, b)
        return carry

    lax.fori_loop(0, NCHUNK // NBUF, group, 0)
    for b in range(NBUF):
        wait_gather(b)
    pltpu.sync_copy(acc_sh.at[pl.ds(sid * RPW, RPW)], out_hbm.at[pl.ds(base, RPW)])


_MB = 2000


def _tc_z_body(h_ref, wg_ref, ws_ref, z_ref):
    w = wg_ref[...] + ws_ref[...]
    z_ref[...] = lax.dot_general(
        h_ref[...], w, (((1,), (1,)), ((), ())), preferred_element_type=jnp.float32
    )


def _tc_z(h, Wg, Ws):
    return pl.pallas_call(
        _tc_z_body,
        grid=(N // _MB,),
        in_specs=[
            pl.BlockSpec((_MB, D), lambda i: (i, 0)),
            pl.BlockSpec((D, D), lambda i: (0, 0)),
            pl.BlockSpec((D, D), lambda i: (0, 0)),
        ],
        out_specs=pl.BlockSpec((_MB, D), lambda i: (i, 0)),
        out_shape=jax.ShapeDtypeStruct((N, D), jnp.float32),
    )(h, Wg, Ws)


def _tc_out_body(z_ref, g_ref, wl_ref, b_ref, h_ref):
    a = (
        z_ref[...]
        + lax.dot_general(
            g_ref[...], wl_ref[...] * (1.0 / DEG), (((1,), (1,)), ((), ())),
            preferred_element_type=jnp.float32,
        )
        + b_ref[...]
    )
    h_ref[...] = jnp.where(a > 0, a, jnp.exp(a) - 1.0)


def _tc_out(z, g, Wl, b):
    return pl.pallas_call(
        _tc_out_body,
        grid=(N // _MB,),
        in_specs=[
            pl.BlockSpec((_MB, D), lambda i: (i, 0)),
            pl.BlockSpec((_MB, D), lambda i: (i, 0)),
            pl.BlockSpec((D, D), lambda i: (0, 0)),
            pl.BlockSpec((1, D), lambda i: (0, 0)),
        ],
        out_specs=pl.BlockSpec((_MB, D), lambda i: (i, 0)),
        out_shape=jax.ShapeDtypeStruct((N, D), jnp.float32),
    )(z, g, Wl, b.reshape(1, D))


def kernel(x, edge, Wg0, Wl0, Ws0, b0, Wg1, Wl1, Ws1, b1, Wg2, Wl2, Ws2, b2):
    dst = edge[1]
    idx_pad = jnp.concatenate(
        [dst, jnp.zeros((NPAD - N) * DEG, dtype=jnp.int32)]
    ).reshape(NW, RPW * DEG)
    idx = jnp.zeros((NW, PER_W), dtype=jnp.int32).at[:, : RPW * DEG].set(idx_pad)
    accidx = (
        jnp.arange(16, dtype=jnp.int32)[:, None, None] * RPW
        + jnp.arange(NCHUNK, dtype=jnp.int32)[None, :, None] * C
        + (jnp.arange(C * DEG, dtype=jnp.int32) // DEG)[None, None, :]
    )
    h = x
    for Wg, Wl, Ws, b in ((Wg0, Wl0, Ws0, b0), (Wg1, Wl1, Ws1, b1), (Wg2, Wl2, Ws2, b2)):
        g = _sc_gather_sum(h, idx, accidx)[:N]
        z = _tc_z(h, Wg, Ws)
        h = _tc_out(z, g, Wl, b)
    return h


# speedup vs baseline: 2.0218x; 2.0218x over previous
import functools

import jax
import jax.numpy as jnp
from jax import lax
from jax.experimental import pallas as pl
from jax.experimental.pallas import tpu as pltpu
from jax.experimental.pallas import tpu_sc as plsc

N = 10000
DEG = 32
D = 128

NW = 32
RPW = 320
NPAD = NW * RPW
C = 4
NCHUNK = RPW // C
NBUF = 2
PER_W = (RPW + NBUF * C) * DEG

_MESH = plsc.VectorSubcoreMesh(core_axis_name="c", subcore_axis_name="s")


@functools.partial(
    pl.kernel,
    mesh=_MESH,
    out_type=jax.ShapeDtypeStruct((NPAD, D), jnp.float32),
    scratch_types=[
        pltpu.VMEM((PER_W,), jnp.int32),
        pltpu.VMEM((NBUF, C * DEG, D), jnp.float32),
        pltpu.VMEM((C, D), jnp.float32),
        pltpu.SemaphoreType.DMA((NBUF,)),
    ],
)
def _sc_gather_sum(table, idx_hbm, out_hbm, idx_v, rows_v, outb_v, gsem):
    wid = lax.axis_index("s") * 2 + lax.axis_index("c")
    base = wid * RPW
    pltpu.sync_copy(idx_hbm.at[wid], idx_v)

    def start_gather(ci, b):
        pltpu.async_copy(
            table.at[idx_v.at[pl.ds(ci * (C * DEG), C * DEG)]],
            rows_v.at[b],
            gsem.at[b],
        )

    def wait_gather(b):
        pltpu.make_async_copy(
            table.at[idx_v.at[pl.ds(0, C * DEG)]], rows_v.at[b], gsem.at[b]
        ).wait()

    for b in range(NBUF):
        start_gather(b, b)

    def group(gi, carry):
        for b in range(NBUF):
            ci = gi * NBUF + b
            wait_gather(b)
            for r in range(C):
                ro = r * DEG
                for j in range(D // 16):
                    sl = pl.ds(j * 16, 16)
                    acc = rows_v[b, ro, sl]
                    for k in range(1, DEG):
                        acc = acc + rows_v[b, ro + k, sl]
                    outb_v[r, sl] = acc
            pltpu.sync_copy(outb_v, out_hbm.at[pl.ds(base + ci * C, C)])
            start_gather(ci + NBUF, b)
        return carry

    lax.fori_loop(0, NCHUNK // NBUF, group, 0)
    for b in range(NBUF):
        wait_gather(b)


_MB = 2000


def _tc_z_body(h_ref, wg_ref, ws_ref, z_ref):
    w = wg_ref[...] + ws_ref[...]
    z_ref[...] = lax.dot_general(
        h_ref[...], w, (((1,), (1,)), ((), ())), preferred_element_type=jnp.float32
    )


def _tc_z(h, Wg, Ws):
    return pl.pallas_call(
        _tc_z_body,
        grid=(N // _MB,),
        in_specs=[
            pl.BlockSpec((_MB, D), lambda i: (i, 0)),
            pl.BlockSpec((D, D), lambda i: (0, 0)),
            pl.BlockSpec((D, D), lambda i: (0, 0)),
        ],
        out_specs=pl.BlockSpec((_MB, D), lambda i: (i, 0)),
        out_shape=jax.ShapeDtypeStruct((N, D), jnp.float32),
    )(h, Wg, Ws)


def _tc_out_body(z_ref, g_ref, wl_ref, b_ref, h_ref):
    a = (
        z_ref[...]
        + lax.dot_general(
            g_ref[...], wl_ref[...] * (1.0 / DEG), (((1,), (1,)), ((), ())),
            preferred_element_type=jnp.float32,
        )
        + b_ref[...]
    )
    h_ref[...] = jnp.where(a > 0, a, jnp.exp(a) - 1.0)


def _tc_out(z, g, Wl, b):
    return pl.pallas_call(
        _tc_out_body,
        grid=(N // _MB,),
        in_specs=[
            pl.BlockSpec((_MB, D), lambda i: (i, 0)),
            pl.BlockSpec((_MB, D), lambda i: (i, 0)),
            pl.BlockSpec((D, D), lambda i: (0, 0)),
            pl.BlockSpec((1, D), lambda i: (0, 0)),
        ],
        out_specs=pl.BlockSpec((_MB, D), lambda i: (i, 0)),
        out_shape=jax.ShapeDtypeStruct((N, D), jnp.float32),
    )(z, g, Wl, b.reshape(1, D))


def kernel(x, edge, Wg0, Wl0, Ws0, b0, Wg1, Wl1, Ws1, b1, Wg2, Wl2, Ws2, b2):
    dst = edge[1]
    idx_pad = jnp.concatenate(
        [dst, jnp.zeros((NPAD - N) * DEG, dtype=jnp.int32)]
    ).reshape(NW, RPW * DEG)
    idx = jnp.zeros((NW, PER_W), dtype=jnp.int32).at[:, : RPW * DEG].set(idx_pad)
    h = x
    for Wg, Wl, Ws, b in ((Wg0, Wl0, Ws0, b0), (Wg1, Wl1, Ws1, b1), (Wg2, Wl2, Ws2, b2)):
        g = _sc_gather_sum(h, idx)[:N]
        z = _tc_z(h, Wg, Ws)
        h = _tc_out(z, g, Wl, b)
    return h
